# baseline (speedup 1.0000x reference)
import dataclasses
import functools

import jax
import jax.numpy as jnp
from jax.experimental import pallas as pl
from jax.experimental.pallas import tpu as pltpu
from jax.experimental.pallas import tpu_sc as plsc

_GROUPS = (
    (0, 1, 2, 8, 25),
    (3, 24),
    (6, 7),
    (4, 9),
    (12, 13, 14, 15),
    (16, 17, 18, 19, 20, 21, 22, 23),
)

_N_COLS = 26
_N_OUT = 6
_LANES = 16
_BLOCK_W = 1536
_TILE = 128


def _block_body(in_vmem, out_vmem):
    @plsc.parallel_loop(0, in_vmem.shape[1] // _LANES, unroll=8)
    def _(i):
        sl = pl.ds(i * _LANES, _LANES)
        for g, idx in enumerate(_GROUPS):
            m = in_vmem[idx[0], sl]
            for c in idx[1:]:
                m = jnp.maximum(m, in_vmem[c, sl])
            out_vmem[g, sl] = m


def kernel(activation):
    n_rows = activation.shape[0]
    n_tiles = -(-n_rows // _TILE)
    n_main = (n_tiles * _TILE) // _BLOCK_W
    tail_t0 = n_main * (_BLOCK_W // _TILE)
    n_tail = n_tiles - tail_t0
    act_t = activation.T
    mesh = plsc.VectorSubcoreMesh(core_axis_name="c", subcore_axis_name="s")
    cp = pltpu.CompilerParams()
    if "needs_layout_passes" in pltpu.CompilerParams.__dataclass_fields__:
        cp = dataclasses.replace(cp, needs_layout_passes=False)
    if "use_tc_tiling_on_sc" in pltpu.CompilerParams.__dataclass_fields__:
        cp = dataclasses.replace(cp, use_tc_tiling_on_sc=True)

    @functools.partial(
        pl.kernel,
        out_type=jax.ShapeDtypeStruct((_N_OUT, n_rows), jnp.float32),
        mesh=mesh,
        compiler_params=cp,
    )
    def run(in_hbm, out_hbm):
        pltpu.emit_pipeline(
            _block_body,
            grid=(n_main,),
            in_specs=[pl.BlockSpec((_N_COLS, _BLOCK_W), lambda i: (0, i))],
            out_specs=[pl.BlockSpec((_N_OUT, _BLOCK_W), lambda i: (0, i))],
            core_axis_name=("c", "s"),
            dimension_semantics=(pltpu.PARALLEL,),
        )(in_hbm, out_hbm)
        if n_tail:
            pltpu.emit_pipeline(
                _block_body,
                grid=(n_tail,),
                in_specs=[
                    pl.BlockSpec((_N_COLS, _TILE), lambda i: (0, i + tail_t0))
                ],
                out_specs=[
                    pl.BlockSpec((_N_OUT, _TILE), lambda i: (0, i + tail_t0))
                ],
                core_axis_name=("c", "s"),
                dimension_semantics=(pltpu.PARALLEL,),
            )(in_hbm, out_hbm)

    return run(act_t).T


# speedup vs baseline: 1.6551x; 1.6551x over previous
import functools

import jax
import jax.numpy as jnp
from jax.experimental import pallas as pl
from jax.experimental.pallas import tpu as pltpu

_GROUPS = (
    (0, 1, 2, 8, 25),
    (3, 24),
    (6, 7),
    (4, 9),
    (12, 13, 14, 15),
    (16, 17, 18, 19, 20, 21, 22, 23),
)

_N_COLS = 26
_N_OUT = 6
_BLOCK_W = 65536


def _tc_body(x_ref, o_ref):
    for g, idx in enumerate(_GROUPS):
        m = x_ref[idx[0], :]
        for c in idx[1:]:
            m = jnp.maximum(m, x_ref[c, :])
        o_ref[g, :] = m


def kernel(activation):
    n_rows = activation.shape[0]
    act_t = activation.T
    n_blocks = -(-n_rows // _BLOCK_W)
    out_t = pl.pallas_call(
        _tc_body,
        grid=(n_blocks,),
        in_specs=[pl.BlockSpec((_N_COLS, _BLOCK_W), lambda i: (0, i))],
        out_specs=pl.BlockSpec((_N_OUT, _BLOCK_W), lambda i: (0, i)),
        out_shape=jax.ShapeDtypeStruct((_N_OUT, n_rows), jnp.float32),
    )(act_t)
    return out_t.T
